# baseline (speedup 1.0000x reference)
import jax
import jax.numpy as jnp
from jax import lax
from jax.experimental import pallas as pl
from jax.experimental.pallas import tpu as pltpu
from jax.experimental.pallas import tpu_sc as plsc

N = 10000
E = 160000
H = 16
G = 64
NC = 2
NS = 16
NW = NC * NS
EPW = E // NW
CH = 1000
NCH = EPW // CH
PADH = 5008
NP = 2 * PADH
NPC2 = NP // NS
UPS = PADH // NS

TN = 2000
TE = 6400
TS = 1600
SUB = TE // TS
HRT = NP // 4
TP = NP // 4

_MAGA = 0x5AC0FFE0
_MAGB = 0x600DCAF0


def _mesh():
    return plsc.VectorSubcoreMesh(core_axis_name="c", subcore_axis_name="s")


def _sc_params():
    return pltpu.CompilerParams(use_tc_tiling_on_sc=False)


def _gpad_body(h_hbm, idx_hbm, hp_hbm, xj_hbm, idx_v, rows_v, pad_v, sem):
    c = lax.axis_index("c")
    s = lax.axis_index("s")
    wid = s * NC + c

    @pl.when(s < 8)
    def _():
        pltpu.sync_copy(h_hbm.at[pl.ds(c * (N // 2) + s * 625, 625)], pad_v)
        pltpu.sync_copy(pad_v, hp_hbm.at[pl.ds(c * PADH + s * 625, 625)])

    pltpu.sync_copy(idx_hbm.at[pl.ds(wid * NCH, NCH)], idx_v)

    def chunk(j, carry):
        pltpu.async_copy(h_hbm.at[idx_v.at[j]],
                         rows_v.at[pl.ds(j * CH, CH)], sem).wait()
        return carry

    lax.fori_loop(0, NCH, chunk, 0)
    pltpu.sync_copy(rows_v, xj_hbm.at[pl.ds(wid * EPW, EPW)])


def _sc_gather_pad(h, idx2d):
    f = pl.kernel(_gpad_body, mesh=_mesh(), compiler_params=_sc_params(),
                  out_type=[jax.ShapeDtypeStruct((NP, H), jnp.float32),
                            jax.ShapeDtypeStruct((E + 16, H), jnp.float32)],
                  scratch_types=[pltpu.VMEM((NCH, CH), jnp.int32),
                                 pltpu.VMEM((EPW, H), jnp.float32),
                                 pltpu.VMEM((625, H), jnp.float32),
                                 pltpu.SemaphoreType.DMA])
    return f(h, idx2d)


NCHH = 5
AGR = PADH + 16
NPC3 = AGR // NS


def _mask_loop(idx_v, eidx_v, c, s, r, lanes, mode):
    half0 = c * (N // 2)
    nblk = CH // 16
    for j in range(NCHH):
        ebase = s * (2 * EPW) + r * EPW + j * CH

        def blk(k, carry, j=j, ebase=ebase):
            v = idx_v[j, pl.ds(k * 16, 16)]
            ok = (v >= half0) & (v < half0 + N // 2)
            if mode == "scatter":
                idx_v[j, pl.ds(k * 16, 16)] = jnp.where(
                    ok, v - half0, PADH + s)
            else:
                idx_v[j, pl.ds(k * 16, 16)] = jnp.where(
                    ok, v + 8 * c, c * PADH)
                eidx_v[j, pl.ds(k * 16, 16)] = jnp.where(
                    ok, ebase + k * 16 + lanes, E + s)
            return carry

        lax.fori_loop(0, nblk, blk, 0)
        tail = CH - 16
        newlane = 16 - (CH - nblk * 16)
        tm = lanes >= newlane
        v = idx_v[j, pl.ds(tail, 16)]
        ok = (v >= half0) & (v < half0 + N // 2)
        if mode == "scatter":
            idx_v[j, pl.ds(tail, 16)] = jnp.where(
                tm, jnp.where(ok, v - half0, PADH + s), v)
        else:
            idx_v[j, pl.ds(tail, 16)] = jnp.where(
                tm, jnp.where(ok, v + 8 * c, c * PADH), v)
            ev = eidx_v[j, pl.ds(tail, 16)]
            eidx_v[j, pl.ds(tail, 16)] = jnp.where(
                tm, jnp.where(ok, ebase + tail + lanes, E + s), ev)


def _make_fused_body(do_gather):
    def body(msg_hbm, dst_hbm, src_hbm, h_hbm, hr_hbm, z_hbm,
             hn_hbm, xj_hbm,
             idx_v, eidx_v, msg_v, st_a, st_b, hr_v, h_v, agg_sh, sem):
        c = lax.axis_index("c")
        s = lax.axis_index("s")
        lanes = lax.broadcasted_iota(jnp.int32, (16,), 0)
        pltpu.sync_copy(z_hbm.at[pl.ds(s * NPC3, NPC3)],
                        agg_sh.at[pl.ds(s * NPC3, NPC3)])
        plsc.subcore_barrier()
        for r in range(2):
            pltpu.sync_copy(dst_hbm.at[pl.ds(s * 2 * NCHH + r * NCHH, NCHH)],
                            idx_v)
            _mask_loop(idx_v, eidx_v, c, s, r, lanes, "scatter")
            pltpu.sync_copy(
                msg_hbm.at[pl.ds(s * 2 * EPW + r * EPW, EPW)], msg_v)

            def sc_chunk(j, carry):
                pltpu.sync_copy(msg_v.at[pl.ds(j * CH, CH)],
                                agg_sh.at[idx_v.at[j]], add=True)
                return carry

            lax.fori_loop(0, NCHH, sc_chunk, 0)
        plsc.subcore_barrier()
        base = c * PADH + s * UPS
        pltpu.sync_copy(agg_sh.at[pl.ds(s * UPS, UPS)], st_a)
        pltpu.sync_copy(hr_hbm.at[pl.ds(base, UPS)], hr_v)
        pltpu.sync_copy(h_hbm.at[pl.ds(base, UPS)], h_v)

        def upd(r2, carry):
            m = st_a[r2, :] + hr_v[r2, :]
            st_b[r2, :] = jnp.maximum(m, 0.0) + h_v[r2, :]
            return carry

        lax.fori_loop(0, UPS, upd, 0)
        pltpu.sync_copy(st_b, hn_hbm.at[pl.ds(base, UPS)])
        if do_gather:
            plsc.subcore_barrier()
            for r in range(2):
                pltpu.sync_copy(
                    src_hbm.at[pl.ds(s * 2 * NCHH + r * NCHH, NCHH)], idx_v)
                _mask_loop(idx_v, eidx_v, c, s, r, lanes, "gather")

                def g_chunk(j, carry):
                    pltpu.async_copy(hn_hbm.at[idx_v.at[j]],
                                     msg_v.at[pl.ds(j * CH, CH)], sem).wait()
                    pltpu.sync_copy(msg_v.at[pl.ds(j * CH, CH)],
                                    xj_hbm.at[eidx_v.at[j]])
                    return carry

                lax.fori_loop(0, NCHH, g_chunk, 0)

    return body


def _sc_fused(msg, dst2d, src2d, hpad, hr, zeros_pad, do_gather):
    f = pl.kernel(_make_fused_body(do_gather),
                  mesh=_mesh(), compiler_params=_sc_params(),
                  out_type=[jax.ShapeDtypeStruct((NP, H), jnp.float32),
                            jax.ShapeDtypeStruct((E + 16, H), jnp.float32)],
                  scratch_types=[pltpu.VMEM((NCHH, CH), jnp.int32),
                                 pltpu.VMEM((NCHH, CH), jnp.int32),
                                 pltpu.VMEM((EPW, H), jnp.float32),
                                 pltpu.VMEM((UPS, H), jnp.float32),
                                 pltpu.VMEM((UPS, H), jnp.float32),
                                 pltpu.VMEM((UPS, H), jnp.float32),
                                 pltpu.VMEM((UPS, H), jnp.float32),
                                 pltpu.VMEM_SHARED((AGR, H), jnp.float32),
                                 pltpu.SemaphoreType.DMA])
    return f(msg, dst2d, src2d, hpad, hr, zeros_pad)


def _inproj_body(x_ref, w_ref, b_ref, o_ref):
    o_ref[...] = jnp.dot(x_ref[...], w_ref[...],
                         preferred_element_type=jnp.float32) + b_ref[...]


def _inproj(x, w, b_row):
    return pl.pallas_call(
        _inproj_body,
        grid=(N // TN,),
        in_specs=[pl.BlockSpec((TN, x.shape[1]), lambda i: (i, 0)),
                  pl.BlockSpec((x.shape[1], H), lambda i: (0, 0)),
                  pl.BlockSpec((1, H), lambda i: (0, 0))],
        out_specs=pl.BlockSpec((TN, H), lambda i: (i, 0)),
        out_shape=jax.ShapeDtypeStruct((N, H), jnp.float32))(x, w, b_row)


def _msg_body(ea_ref, xj_ref, w1_ref, w2p_ref, smat_ref, t16_ref,
              h_ref, root_ref, cb_ref, o_ref, hr_ref):
    i = pl.program_id(0)

    @pl.when(i < NP // HRT)
    def _():
        hr_ref[...] = jnp.dot(h_ref[...], root_ref[...],
                              preferred_element_type=jnp.float32) + cb_ref[...]

    w1 = w1_ref[...]
    w2p = w2p_ref[...]
    smat = smat_ref[...]
    t16 = t16_ref[...]
    for st in range(SUB):
        r0 = st * TS
        ea = ea_ref[r0:r0 + TS, :]
        xj = xj_ref[r0:r0 + TS, :].astype(jnp.bfloat16)
        ew = jnp.maximum(
            jnp.dot(ea, w1, preferred_element_type=jnp.float32,
                    precision=lax.Precision.DEFAULT),
            0.0).astype(jnp.bfloat16)
        wp = jnp.dot(ew, w2p, preferred_element_type=jnp.float32,
                     precision=lax.Precision.DEFAULT).astype(jnp.bfloat16)
        xt = jnp.dot(xj, t16, preferred_element_type=jnp.float32,
                     precision=lax.Precision.DEFAULT).astype(jnp.bfloat16)
        o_ref[r0:r0 + TS, :] = jnp.dot(wp * xt, smat,
                                       preferred_element_type=jnp.float32,
                                       precision=lax.Precision.DEFAULT)


def _msg_call(ea_aug, xj, w1a, w2pa, smat_bf, t16_bf, hpad, rootl, cb_row):
    return pl.pallas_call(
        _msg_body,
        grid=(E // TE,),
        in_specs=[pl.BlockSpec((TE, H + 1), lambda i: (i, 0)),
                  pl.BlockSpec((TE, H), lambda i: (i, 0)),
                  pl.BlockSpec((H + 1, 2 * H + 1), lambda i: (0, 0)),
                  pl.BlockSpec((2 * H + 1, H * H), lambda i: (0, 0)),
                  pl.BlockSpec((H * H, H), lambda i: (0, 0)),
                  pl.BlockSpec((H, H * H), lambda i: (0, 0)),
                  pl.BlockSpec((HRT, H), lambda i: (i % (NP // HRT), 0)),
                  pl.BlockSpec((H, H), lambda i: (0, 0)),
                  pl.BlockSpec((1, H), lambda i: (0, 0))],
        out_specs=[pl.BlockSpec((TE, H), lambda i: (i, 0)),
                   pl.BlockSpec((HRT, H), lambda i: (i % (NP // HRT), 0))],
        out_shape=[jax.ShapeDtypeStruct((E, H), jnp.float32),
                   jax.ShapeDtypeStruct((NP, H), jnp.float32)])(
            ea_aug, xj, w1a, w2pa, smat_bf, t16_bf, hpad, rootl, cb_row)


def _pool_body(b_ref, h_ref, w1_ref, b1_ref, w2_ref, b2_ref, o_ref, sums, cnts):
    i = pl.program_id(0)

    @pl.when(i == 0)
    def _init():
        sums[...] = jnp.zeros_like(sums)
        cnts[...] = jnp.zeros_like(cnts)

    b_row = b_ref[0]
    gidx = lax.broadcasted_iota(jnp.int32, (G, 1), 0)
    pt = (b_row == gidx).astype(jnp.float32)
    sums[...] += jnp.dot(pt, h_ref[...], preferred_element_type=jnp.float32)
    cnts[...] += jnp.sum(pt, axis=1, keepdims=True)

    @pl.when(i == pl.num_programs(0) - 1)
    def _finish():
        g = sums[...] / jnp.maximum(cnts[...], 1.0)
        r = jnp.maximum(
            jnp.dot(g, w1_ref[...], preferred_element_type=jnp.float32)
            + b1_ref[...], 0.0)
        o_ref[...] = (jnp.dot(r, w2_ref[...], preferred_element_type=jnp.float32)
                      + b2_ref[...])


def _pool(batch3, hpad, w1, b1_row, w2, b2_row):
    return pl.pallas_call(
        _pool_body,
        grid=(NP // TP,),
        in_specs=[pl.BlockSpec((1, 1, TP), lambda i: (i, 0, 0)),
                  pl.BlockSpec((TP, H), lambda i: (i, 0)),
                  pl.BlockSpec((H, H), lambda i: (0, 0)),
                  pl.BlockSpec((1, H), lambda i: (0, 0)),
                  pl.BlockSpec((H, 1), lambda i: (0, 0)),
                  pl.BlockSpec((1, 1), lambda i: (0, 0))],
        out_specs=pl.BlockSpec((G, 1), lambda i: (0, 0)),
        out_shape=jax.ShapeDtypeStruct((G, 1), jnp.float32),
        scratch_shapes=[pltpu.VMEM((G, H), jnp.float32),
                        pltpu.VMEM((G, 1), jnp.float32)])(
            batch3, hpad, w1, b1_row, w2, b2_row)


def kernel(x, edge_index, edge_attr, batch, W_in, b_in, em_w1, em_b1, em_w2,
           em_b2, root, conv_b, ro_w1, ro_b1, ro_w2, ro_b2):
    bf = jnp.bfloat16
    src2d = edge_index[0].reshape(E // CH, CH)
    dst2d = edge_index[1].reshape(E // CH, CH)
    pad_ids = jnp.full((8,), 2 ** 30, jnp.int32)
    batch3 = jnp.concatenate(
        [batch[:N // 2], pad_ids, batch[N // 2:], pad_ids]).reshape(NP // TP, 1, TP)
    zeros_pad = jnp.zeros((NP, H), jnp.float32)
    ea_aug = jnp.concatenate(
        [edge_attr, jnp.ones((E, 1), jnp.float32)], axis=1).astype(bf)
    sj = jnp.arange(H * H)[:, None]
    smat_bf = (sj // H == jnp.arange(H)[None, :]).astype(bf)
    t16_bf = (jnp.arange(H * H)[None, :] % H == jnp.arange(H)[:, None]).astype(bf)

    h0 = _inproj(x, W_in, b_in.reshape(1, H))
    hpad, xj = _sc_gather_pad(h0, src2d)
    L = em_w1.shape[0]
    for l in range(L):
        w2p = em_w2[l].reshape(2 * H, H, H).transpose(0, 2, 1).reshape(2 * H, H * H)
        b2p = em_b2[l].reshape(H, H).T.reshape(1, H * H)
        w1a = jnp.zeros((H + 1, 2 * H + 1), jnp.float32)
        w1a = w1a.at[:H, :2 * H].set(em_w1[l]).at[H, :2 * H].set(em_b1[l])
        w1a = w1a.at[H, 2 * H].set(1.0).astype(bf)
        w2pa = jnp.concatenate([w2p, b2p], axis=0).astype(bf)
        msg, hr = _msg_call(ea_aug, xj, w1a, w2pa, smat_bf, t16_bf,
                            hpad, root[l], conv_b[l].reshape(1, H))
        hpad, xj = _sc_fused(msg, dst2d, src2d, hpad, hr, zeros_pad,
                             do_gather=(l < L - 1))
    return _pool(batch3, hpad, ro_w1, ro_b1.reshape(1, H),
                 ro_w2, ro_b2.reshape(1, 1))


# speedup vs baseline: 2.0498x; 2.0498x over previous
import jax
import jax.numpy as jnp
from jax import lax
from jax.experimental import pallas as pl
from jax.experimental.pallas import tpu as pltpu
from jax.experimental.pallas import tpu_sc as plsc

N = 10000
E = 160000
H = 16
G = 64
NC = 2
NS = 16
NW = NC * NS
EPW = E // NW
CH = 1000
NCH = EPW // CH
NPC = N // NS

TN = 2000
TE = 6400
TS = 1600
SUB = TE // TS

def _mesh():
    return plsc.VectorSubcoreMesh(core_axis_name="c", subcore_axis_name="s")


def _gather_body(h_hbm, idx_hbm, out_hbm, idx_v, rows_v, sem):
    c = lax.axis_index("c")
    s = lax.axis_index("s")
    wid = s * NC + c
    pltpu.sync_copy(idx_hbm.at[pl.ds(wid * NCH, NCH)], idx_v)

    def chunk(j, carry):
        pltpu.async_copy(h_hbm.at[idx_v.at[j]],
                         rows_v.at[pl.ds(j * CH, CH)], sem).wait()
        return carry

    lax.fori_loop(0, NCH, chunk, 0)
    pltpu.sync_copy(rows_v, out_hbm.at[pl.ds(wid * EPW, EPW)])


def _sc_gather(h, idx2d):
    f = pl.kernel(_gather_body, mesh=_mesh(),
                  compiler_params=pltpu.CompilerParams(use_tc_tiling_on_sc=False),
                  out_type=jax.ShapeDtypeStruct((E, H), jnp.float32),
                  scratch_types=[pltpu.VMEM((NCH, CH), jnp.int32),
                                 pltpu.VMEM((EPW, H), jnp.float32),
                                 pltpu.SemaphoreType.DMA])
    return f(h, idx2d)


def _scatter_body(msg_hbm, idx_hbm, zeros_hbm, out_hbm, idx_v, msg_v, agg_sh, sem):
    c = lax.axis_index("c")
    s = lax.axis_index("s")
    wid = s * NC + c
    pltpu.sync_copy(zeros_hbm.at[pl.ds(s * NPC, NPC)],
                    agg_sh.at[pl.ds(s * NPC, NPC)])
    pltpu.sync_copy(idx_hbm.at[pl.ds(wid * NCH, NCH)], idx_v)
    pltpu.sync_copy(msg_hbm.at[pl.ds(wid * EPW, EPW)], msg_v)
    plsc.subcore_barrier()

    def chunk(j, carry):
        pltpu.sync_copy(msg_v.at[pl.ds(j * CH, CH)],
                        agg_sh.at[idx_v.at[j]], add=True)
        return carry

    lax.fori_loop(0, NCH, chunk, 0)
    plsc.subcore_barrier()
    pltpu.sync_copy(agg_sh.at[pl.ds(s * NPC, NPC)],
                    out_hbm.at[c].at[pl.ds(s * NPC, NPC)])


def _sc_scatter(msg, idx2d, zeros_n):
    f = pl.kernel(_scatter_body, mesh=_mesh(),
                  compiler_params=pltpu.CompilerParams(use_tc_tiling_on_sc=False),
                  out_type=jax.ShapeDtypeStruct((NC, N, H), jnp.float32),
                  scratch_types=[pltpu.VMEM((NCH, CH), jnp.int32),
                                 pltpu.VMEM((EPW, H), jnp.float32),
                                 pltpu.VMEM_SHARED((N, H), jnp.float32),
                                 pltpu.SemaphoreType.DMA])
    return f(msg, idx2d, zeros_n)


def _inproj_body(x_ref, w_ref, b_ref, o_ref):
    o_ref[...] = jnp.dot(x_ref[...], w_ref[...],
                         preferred_element_type=jnp.float32) + b_ref[...]


def _inproj(x, w, b_row):
    return pl.pallas_call(
        _inproj_body,
        grid=(N // TN,),
        in_specs=[pl.BlockSpec((TN, x.shape[1]), lambda i: (i, 0)),
                  pl.BlockSpec((x.shape[1], H), lambda i: (0, 0)),
                  pl.BlockSpec((1, H), lambda i: (0, 0))],
        out_specs=pl.BlockSpec((TN, H), lambda i: (i, 0)),
        out_shape=jax.ShapeDtypeStruct((N, H), jnp.float32))(x, w, b_row)


def _msg_body(ea_ref, xj_ref, w1_ref, w2p_ref, smat_ref, t16_ref, o_ref):
    w1 = w1_ref[...]
    w2p = w2p_ref[...]
    smat = smat_ref[...]
    t16 = t16_ref[...]
    for st in range(SUB):
        r0 = st * TS
        ea = ea_ref[r0:r0 + TS, :]
        xj = xj_ref[r0:r0 + TS, :].astype(jnp.bfloat16)
        ew = jnp.maximum(
            jnp.dot(ea, w1, preferred_element_type=jnp.float32), 0.0)
        wp = jnp.dot(ew, w2p,
                     preferred_element_type=jnp.float32).astype(jnp.bfloat16)
        xt = jnp.dot(xj, t16, preferred_element_type=jnp.float32,
                     precision=lax.Precision.DEFAULT).astype(jnp.bfloat16)
        o_ref[r0:r0 + TS, :] = jnp.dot(wp * xt, smat,
                                       preferred_element_type=jnp.float32,
                                       precision=lax.Precision.DEFAULT)


def _msg_call(ea_aug, xj, w1a, w2pa, smat_bf, t16_bf):
    return pl.pallas_call(
        _msg_body,
        grid=(E // TE,),
        in_specs=[pl.BlockSpec((TE, H + 1), lambda i: (i, 0)),
                  pl.BlockSpec((TE, H), lambda i: (i, 0)),
                  pl.BlockSpec((H + 1, 2 * H + 1), lambda i: (0, 0)),
                  pl.BlockSpec((2 * H + 1, H * H), lambda i: (0, 0)),
                  pl.BlockSpec((H * H, H), lambda i: (0, 0)),
                  pl.BlockSpec((H, H * H), lambda i: (0, 0))],
        out_specs=pl.BlockSpec((TE, H), lambda i: (i, 0)),
        out_shape=jax.ShapeDtypeStruct((E, H), jnp.float32))(
            ea_aug, xj, w1a, w2pa, smat_bf, t16_bf)


def _update_body(h_ref, a0_ref, a1_ref, r_ref, b_ref, o_ref):
    h = h_ref[...]
    m = (a0_ref[...] + a1_ref[...]
         + jnp.dot(h, r_ref[...], preferred_element_type=jnp.float32)
         + b_ref[...])
    o_ref[...] = jnp.maximum(m, 0.0) + h


def _update(h, a0, a1, rootl, b_row):
    return pl.pallas_call(
        _update_body,
        grid=(N // TN,),
        in_specs=[pl.BlockSpec((TN, H), lambda i: (i, 0)),
                  pl.BlockSpec((TN, H), lambda i: (i, 0)),
                  pl.BlockSpec((TN, H), lambda i: (i, 0)),
                  pl.BlockSpec((H, H), lambda i: (0, 0)),
                  pl.BlockSpec((1, H), lambda i: (0, 0))],
        out_specs=pl.BlockSpec((TN, H), lambda i: (i, 0)),
        out_shape=jax.ShapeDtypeStruct((N, H), jnp.float32))(
            h, a0, a1, rootl, b_row)


def _pool_body(b_ref, h_ref, w1_ref, b1_ref, w2_ref, b2_ref, o_ref, sums, cnts):
    i = pl.program_id(0)

    @pl.when(i == 0)
    def _init():
        sums[...] = jnp.zeros_like(sums)
        cnts[...] = jnp.zeros_like(cnts)

    b_row = b_ref[0]
    gidx = lax.broadcasted_iota(jnp.int32, (G, 1), 0)
    pt = (b_row == gidx).astype(jnp.float32)
    sums[...] += jnp.dot(pt, h_ref[...], preferred_element_type=jnp.float32)
    cnts[...] += jnp.sum(pt, axis=1, keepdims=True)

    @pl.when(i == pl.num_programs(0) - 1)
    def _finish():
        g = sums[...] / jnp.maximum(cnts[...], 1.0)
        r = jnp.maximum(
            jnp.dot(g, w1_ref[...], preferred_element_type=jnp.float32)
            + b1_ref[...], 0.0)
        o_ref[...] = (jnp.dot(r, w2_ref[...], preferred_element_type=jnp.float32)
                      + b2_ref[...])


def _pool(batch3, h, w1, b1_row, w2, b2_row):
    return pl.pallas_call(
        _pool_body,
        grid=(N // TN,),
        in_specs=[pl.BlockSpec((1, 1, TN), lambda i: (i, 0, 0)),
                  pl.BlockSpec((TN, H), lambda i: (i, 0)),
                  pl.BlockSpec((H, H), lambda i: (0, 0)),
                  pl.BlockSpec((1, H), lambda i: (0, 0)),
                  pl.BlockSpec((H, 1), lambda i: (0, 0)),
                  pl.BlockSpec((1, 1), lambda i: (0, 0))],
        out_specs=pl.BlockSpec((G, 1), lambda i: (0, 0)),
        out_shape=jax.ShapeDtypeStruct((G, 1), jnp.float32),
        scratch_shapes=[pltpu.VMEM((G, H), jnp.float32),
                        pltpu.VMEM((G, 1), jnp.float32)])(
            batch3, h, w1, b1_row, w2, b2_row)


def kernel(x, edge_index, edge_attr, batch, W_in, b_in, em_w1, em_b1, em_w2,
           em_b2, root, conv_b, ro_w1, ro_b1, ro_w2, ro_b2):
    src2d = edge_index[0].reshape(E // CH, CH)
    dst2d = edge_index[1].reshape(E // CH, CH)
    batch3 = batch.reshape(N // TN, 1, TN)
    zeros_n = jnp.zeros((N, H), jnp.float32)

    bf = jnp.bfloat16
    ea_aug = jnp.concatenate(
        [edge_attr, jnp.ones((E, 1), jnp.float32)], axis=1)
    sj = jnp.arange(H * H)[:, None]
    smat_bf = (sj // H == jnp.arange(H)[None, :]).astype(bf)
    t16_bf = (jnp.arange(H * H)[None, :] % H == jnp.arange(H)[:, None]).astype(bf)
    h = _inproj(x, W_in, b_in.reshape(1, H))
    for l in range(em_w1.shape[0]):
        w2p = em_w2[l].reshape(2 * H, H, H).transpose(0, 2, 1).reshape(2 * H, H * H)
        b2p = em_b2[l].reshape(H, H).T.reshape(1, H * H)
        w1a = jnp.zeros((H + 1, 2 * H + 1), jnp.float32)
        w1a = w1a.at[:H, :2 * H].set(em_w1[l]).at[H, :2 * H].set(em_b1[l])
        w1a = w1a.at[H, 2 * H].set(1.0)
        w2pa = jnp.concatenate([w2p, b2p], axis=0)
        xj = _sc_gather(h, src2d)
        msg = _msg_call(ea_aug, xj, w1a, w2pa, smat_bf, t16_bf)
        aggp = _sc_scatter(msg, dst2d, zeros_n)
        h = _update(h, aggp[0], aggp[1], root[l], conv_b[l].reshape(1, H))
    return _pool(batch3, h, ro_w1, ro_b1.reshape(1, H),
                 ro_w2, ro_b2.reshape(1, 1))
